# speedup vs baseline: 1.1420x; 1.1420x over previous
import functools

import jax
import jax.numpy as jnp
from jax import lax
from jax.experimental import pallas as pl
from jax.experimental.pallas import tpu as pltpu
from jax.experimental.pallas import tpu_sc as plsc

VOCAB = 104353
DIM = 128
B = 4096
L = 50

NC = 2
NS = 16
NW = NC * NS

TOTAL = B * L
PER_W = TOTAL // NW
CHUNK = 128
NCHUNK = PER_W // CHUNK


def _make_gather():
    mesh = plsc.VectorSubcoreMesh(core_axis_name="c", subcore_axis_name="s")

    @functools.partial(
        pl.kernel,
        mesh=mesh,
        out_type=jax.ShapeDtypeStruct((TOTAL, DIM), jnp.float32),
        scratch_types=[
            pltpu.VMEM((NCHUNK, CHUNK), jnp.int32),
            pltpu.VMEM((CHUNK, DIM), jnp.float32),
            pltpu.SemaphoreType.DMA,
        ],
    )
    def gather_kernel(x_hbm, table_hbm, out_hbm, idx_v, rows_v, sem):
        wid = lax.axis_index("s") * NC + lax.axis_index("c")
        base = wid * PER_W
        pltpu.sync_copy(x_hbm.at[wid], idx_v)

        def body(j, carry):
            pltpu.async_copy(table_hbm.at[idx_v.at[j]], rows_v, sem).wait()
            pltpu.sync_copy(rows_v, out_hbm.at[pl.ds(base + j * CHUNK, CHUNK)])
            return carry

        lax.fori_loop(0, NCHUNK, body, 0)

    return gather_kernel


_gather = _make_gather()


@jax.jit
def kernel(x, table):
    xf = x.reshape(NW, NCHUNK, CHUNK)
    out = _gather(xf, table)
    return out.reshape(B, L, DIM)


# speedup vs baseline: 1.2675x; 1.1099x over previous
import functools

import jax
import jax.numpy as jnp
from jax import lax
from jax.experimental import pallas as pl
from jax.experimental.pallas import tpu as pltpu
from jax.experimental.pallas import tpu_sc as plsc

VOCAB = 104353
DIM = 128
B = 4096
L = 50

NC = 2
NS = 16
NW = NC * NS

TOTAL = B * L
PER_W = TOTAL // NW
CHUNK = 128
NCHUNK = PER_W // CHUNK
NBUF = 5
NG = NCHUNK // NBUF


def _make_gather():
    mesh = plsc.VectorSubcoreMesh(core_axis_name="c", subcore_axis_name="s")

    @functools.partial(
        pl.kernel,
        mesh=mesh,
        out_type=jax.ShapeDtypeStruct((TOTAL, DIM), jnp.float32),
        scratch_types=[
            pltpu.VMEM((NCHUNK, CHUNK), jnp.int32),
            *[pltpu.VMEM((CHUNK, DIM), jnp.float32) for _ in range(NBUF)],
            pltpu.SemaphoreType.DMA((NBUF,)),
            pltpu.SemaphoreType.DMA((NBUF,)),
        ],
    )
    def gather_kernel(x_hbm, table_hbm, out_hbm, idx_v, *bufs_and_sems):
        bufs = bufs_and_sems[:NBUF]
        gsem, wsem = bufs_and_sems[NBUF], bufs_and_sems[NBUF + 1]
        wid = lax.axis_index("s") * NC + lax.axis_index("c")
        base = wid * PER_W
        pltpu.sync_copy(x_hbm.at[wid], idx_v)

        def gather_start(j, b):
            pltpu.make_async_copy(
                table_hbm.at[idx_v.at[j]], bufs[b], gsem.at[b]
            ).start()

        def wb_start(j, b):
            pltpu.make_async_copy(
                bufs[b], out_hbm.at[pl.ds(base + j * CHUNK, CHUNK)], wsem.at[b]
            ).start()

        def gather_wait(j, b):
            pltpu.make_async_copy(
                table_hbm.at[idx_v.at[j]], bufs[b], gsem.at[b]
            ).wait()

        def wb_wait(j, b):
            pltpu.make_async_copy(
                bufs[b], out_hbm.at[pl.ds(base + j * CHUNK, CHUNK)], wsem.at[b]
            ).wait()

        for b in range(NBUF):
            gather_start(b, b)

        def body(g, carry):
            for b in range(NBUF):
                j = g * NBUF + b
                gather_wait(j, b)
                wb_start(j, b)
            for b in range(NBUF):
                j = g * NBUF + b
                wb_wait(j, b)
                gather_start(j + NBUF, b)
            return carry

        lax.fori_loop(0, NG - 1, body, 0)

        for b in range(NBUF):
            j = (NG - 1) * NBUF + b
            gather_wait(j, b)
            wb_start(j, b)
        for b in range(NBUF):
            j = (NG - 1) * NBUF + b
            wb_wait(j, b)

    return gather_kernel


_gather = _make_gather()


@jax.jit
def kernel(x, table):
    xf = x.reshape(NW, NCHUNK, CHUNK)
    out = _gather(xf, table)
    return out.reshape(B, L, DIM)


# speedup vs baseline: 2.2748x; 1.7948x over previous
import functools

import jax
import jax.numpy as jnp
from jax import lax
from jax.experimental import pallas as pl
from jax.experimental.pallas import tpu as pltpu
from jax.experimental.pallas import tpu_sc as plsc

VOCAB = 104353
DIM = 128
B = 4096
L = 50

NC = 2
NS = 16
NW = NC * NS

PER_B = B // NW
NBUF = 8
NG = PER_B // NBUF


def _make_gather():
    mesh = plsc.VectorSubcoreMesh(core_axis_name="c", subcore_axis_name="s")

    @functools.partial(
        pl.kernel,
        mesh=mesh,
        out_type=jax.ShapeDtypeStruct((B, L, DIM), jnp.float32),
        scratch_types=[
            pltpu.VMEM((PER_B, L), jnp.int32),
            *[pltpu.VMEM((1, L, DIM), jnp.float32) for _ in range(NBUF)],
            pltpu.SemaphoreType.DMA((NBUF,)),
            pltpu.SemaphoreType.DMA((NBUF,)),
        ],
    )
    def gather_kernel(x_hbm, table_hbm, out_hbm, idx_v, *bufs_and_sems):
        bufs = bufs_and_sems[:NBUF]
        gsem, wsem = bufs_and_sems[NBUF], bufs_and_sems[NBUF + 1]
        wid = lax.axis_index("s") * NC + lax.axis_index("c")
        base = wid * PER_B
        pltpu.sync_copy(x_hbm.at[wid], idx_v)

        def gather_start(j, r):
            pltpu.make_async_copy(
                table_hbm.at[idx_v.at[j]], bufs[r].at[0], gsem.at[r]
            ).start()

        def gather_wait(j, r):
            pltpu.make_async_copy(
                table_hbm.at[idx_v.at[j]], bufs[r].at[0], gsem.at[r]
            ).wait()

        def wb_start(j, r):
            pltpu.make_async_copy(
                bufs[r], out_hbm.at[pl.ds(base + j, 1)], wsem.at[r]
            ).start()

        def wb_wait(j, r):
            pltpu.make_async_copy(
                bufs[r], out_hbm.at[pl.ds(base + j, 1)], wsem.at[r]
            ).wait()

        for r in range(NBUF):
            gather_start(r, r)

        def body(g, carry):
            for r in range(NBUF):
                j = g * NBUF + r
                gather_wait(j, r)
                wb_start(j, r)
            for r in range(NBUF):
                j = g * NBUF + r
                wb_wait(j, r)
                gather_start(j + NBUF, r)
            return carry

        lax.fori_loop(0, NG - 1, body, 0)

        for r in range(NBUF):
            j = (NG - 1) * NBUF + r
            gather_wait(j, r)
            wb_start(j, r)
        for r in range(NBUF):
            j = (NG - 1) * NBUF + r
            wb_wait(j, r)

    return gather_kernel


_gather = _make_gather()


@jax.jit
def kernel(x, table):
    xf = x.reshape(NW, PER_B, L)
    return _gather(xf, table)
